# baseline (speedup 1.0000x reference)
import functools

import jax
import jax.numpy as jnp
from jax import lax
from jax.experimental import pallas as pl
from jax.experimental.pallas import tpu as pltpu
from jax.experimental.pallas import tpu_sc as plsc

_B_TOK = 16384
_T = 50
_D = 64
_B = _B_TOK * _T
_NC = 2
_NS = 16
_NW = _NC * _NS
_B_PER_W = _B // _NW
_CHUNK = 512
_N_CHUNKS = _B_PER_W // _CHUNK
_L = 16

_mesh = plsc.VectorSubcoreMesh(core_axis_name="c", subcore_axis_name="s")


@functools.partial(
    pl.kernel,
    mesh=_mesh,
    out_type=jax.ShapeDtypeStruct((_T, _D, _B_TOK), jnp.float32),
    scratch_types=[
        pltpu.VMEM((_CHUNK,), jnp.int32),
        pltpu.VMEM((_CHUNK,), jnp.int32),
        pltpu.VMEM((_CHUNK, _D), jnp.float32),
        pltpu.VMEM((_CHUNK, _D), jnp.float32),
        pltpu.VMEM((_D, _CHUNK), jnp.float32),
        pltpu.SemaphoreType.DMA,
        pltpu.SemaphoreType.DMA,
        pltpu.SemaphoreType.DMA,
    ],
    compiler_params=pltpu.CompilerParams(
        use_tc_tiling_on_sc=False, needs_layout_passes=False),
)
def _gather_rows(idx_hbm, table_hbm, out_hbm,
                 idx0, idx1, rows0, rows1, rows_t, gs0, gs1, ss):
    wid = lax.axis_index("s") * _NC + lax.axis_index("c")
    base = wid * _B_PER_W
    bufs = ((idx0, rows0, gs0), (idx1, rows1, gs1))
    iota = lax.iota(jnp.int32, _L)

    def load_idx(c, idx_v):
        pltpu.sync_copy(idx_hbm.at[pl.ds(base + c * _CHUNK, _CHUNK)], idx_v)

    def fire_gather(idx_v, rows_v, sem):
        pltpu.async_copy(table_hbm.at[idx_v], rows_v, sem)

    def wait_gather(idx_v, rows_v, sem):
        pltpu.make_async_copy(table_hbm.at[idx_v], rows_v, sem).wait()

    def out_slice(c):
        fl = base + c * _CHUNK
        return out_hbm.at[fl // _B_TOK, :, pl.ds(fl % _B_TOK, _CHUNK)]

    def transpose_chunk(rows_v):
        @pl.loop(0, _D)
        def _d(d):
            col = jnp.full((_L,), d, jnp.int32)
            row_t = rows_t.at[d]

            @pl.loop(0, _CHUNK // _L, unroll=8)
            def _g(g):
                r0 = g * _L
                row_t[pl.ds(r0, _L)] = plsc.load_gather(
                    rows_v, [iota + r0, col])

    for b, (idx_v, rows_v, gsem) in enumerate(bufs):
        load_idx(b, idx_v)
        fire_gather(idx_v, rows_v, gsem)

    @pl.loop(0, _N_CHUNKS - 2, step=2)
    def _pair(i):
        for b, (idx_v, rows_v, gsem) in enumerate(bufs):
            c = i + b
            wait_gather(idx_v, rows_v, gsem)

            @pl.when(c > 0)
            def _():
                pltpu.make_async_copy(rows_t, out_slice(c), ss).wait()

            transpose_chunk(rows_v)
            pltpu.async_copy(rows_t, out_slice(c), ss)
            load_idx(c + 2, idx_v)
            fire_gather(idx_v, rows_v, gsem)

    for b, (idx_v, rows_v, gsem) in enumerate(bufs):
        c = _N_CHUNKS - 2 + b
        wait_gather(idx_v, rows_v, gsem)
        pltpu.make_async_copy(rows_t, out_slice(c), ss).wait()
        transpose_chunk(rows_v)
        pltpu.async_copy(rows_t, out_slice(c), ss)
    pltpu.make_async_copy(rows_t, out_slice(_N_CHUNKS - 1), ss).wait()


def kernel(token_ids, weight):
    idx = token_ids.T.ravel().astype(jnp.int32)
    out_p = _gather_rows(idx, weight)
    return jnp.transpose(out_p, (2, 0, 1))


# speedup vs baseline: 1.1352x; 1.1352x over previous
import functools

import jax
import jax.numpy as jnp
from jax import lax
from jax.experimental import pallas as pl
from jax.experimental.pallas import tpu as pltpu
from jax.experimental.pallas import tpu_sc as plsc

_B_TOK = 16384
_T = 50
_D = 64
_B = _B_TOK * _T
_NC = 2
_NS = 16
_NW = _NC * _NS
_B_PER_W = _B // _NW
_CHUNK = 512
_N_CHUNKS = _B_PER_W // _CHUNK
_L = 16

_mesh = plsc.VectorSubcoreMesh(core_axis_name="c", subcore_axis_name="s")


@functools.partial(
    pl.kernel,
    mesh=_mesh,
    out_type=jax.ShapeDtypeStruct((_T, _D, _B_TOK), jnp.float32),
    scratch_types=[
        pltpu.VMEM((_CHUNK,), jnp.int32),
        pltpu.VMEM((_CHUNK,), jnp.int32),
        pltpu.VMEM((_CHUNK, _D), jnp.float32),
        pltpu.VMEM((_CHUNK, _D), jnp.float32),
        pltpu.VMEM((_D, _CHUNK), jnp.float32),
        pltpu.SemaphoreType.DMA,
        pltpu.SemaphoreType.DMA,
        pltpu.SemaphoreType.DMA,
    ],
    compiler_params=pltpu.CompilerParams(
        use_tc_tiling_on_sc=False, needs_layout_passes=False),
)
def _gather_rows(idx_hbm, table_hbm, out_hbm,
                 idx0, idx1, rows0, rows1, rows_t, gs0, gs1, ss):
    wid = lax.axis_index("s") * _NC + lax.axis_index("c")
    base = wid * _B_PER_W
    bufs = ((idx0, rows0, gs0), (idx1, rows1, gs1))
    iota = lax.iota(jnp.int32, _L)

    def load_idx(c, idx_v):
        pltpu.sync_copy(idx_hbm.at[pl.ds(base + c * _CHUNK, _CHUNK)], idx_v)

    def fire_gather(idx_v, rows_v, sem):
        pltpu.async_copy(table_hbm.at[idx_v], rows_v, sem)

    def wait_gather(idx_v, rows_v, sem):
        pltpu.make_async_copy(table_hbm.at[idx_v], rows_v, sem).wait()

    def out_slice(c):
        fl = base + c * _CHUNK
        return out_hbm.at[fl // _B_TOK, :, pl.ds(fl % _B_TOK, _CHUNK)]

    def transpose_chunk(rows_v):
        n_batch = 8
        @pl.loop(0, _D)
        def _d(d):
            col = jnp.full((_L,), d, jnp.int32)
            row_t = rows_t.at[d]

            @pl.loop(0, _CHUNK // (_L * n_batch))
            def _g(g):
                r0 = g * (_L * n_batch)
                vals = [
                    plsc.load_gather(rows_v, [iota + (r0 + k * _L), col])
                    for k in range(n_batch)
                ]
                for k in range(n_batch):
                    row_t[pl.ds(r0 + k * _L, _L)] = vals[k]

    for b, (idx_v, rows_v, gsem) in enumerate(bufs):
        load_idx(b, idx_v)
        fire_gather(idx_v, rows_v, gsem)

    @pl.loop(0, _N_CHUNKS - 2, step=2)
    def _pair(i):
        for b, (idx_v, rows_v, gsem) in enumerate(bufs):
            c = i + b
            wait_gather(idx_v, rows_v, gsem)

            @pl.when(c > 0)
            def _():
                pltpu.make_async_copy(rows_t, out_slice(c), ss).wait()

            transpose_chunk(rows_v)
            pltpu.async_copy(rows_t, out_slice(c), ss)
            load_idx(c + 2, idx_v)
            fire_gather(idx_v, rows_v, gsem)

    for b, (idx_v, rows_v, gsem) in enumerate(bufs):
        c = _N_CHUNKS - 2 + b
        wait_gather(idx_v, rows_v, gsem)
        pltpu.make_async_copy(rows_t, out_slice(c), ss).wait()
        transpose_chunk(rows_v)
        pltpu.async_copy(rows_t, out_slice(c), ss)
    pltpu.make_async_copy(rows_t, out_slice(_N_CHUNKS - 1), ss).wait()


def kernel(token_ids, weight):
    idx = token_ids.T.ravel().astype(jnp.int32)
    out_p = _gather_rows(idx, weight)
    return jnp.transpose(out_p, (2, 0, 1))


# speedup vs baseline: 2.1081x; 1.8570x over previous
import functools

import jax
import jax.numpy as jnp
from jax import lax
from jax.experimental import pallas as pl
from jax.experimental.pallas import tpu as pltpu
from jax.experimental.pallas import tpu_sc as plsc

_B_TOK = 16384
_T = 50
_D = 64
_B = _B_TOK * _T
_NC = 2
_NS = 16
_NW = _NC * _NS
_B_PER_W = _B // _NW
_CHUNK = 512
_N_CHUNKS = _B_PER_W // _CHUNK
_L = 16

_mesh = plsc.VectorSubcoreMesh(core_axis_name="c", subcore_axis_name="s")


@functools.partial(
    pl.kernel,
    mesh=_mesh,
    out_type=jax.ShapeDtypeStruct((_T, _D, _B_TOK), jnp.float32),
    scratch_types=[
        pltpu.VMEM((_CHUNK,), jnp.int32),
        pltpu.VMEM((_CHUNK,), jnp.int32),
        pltpu.VMEM((_CHUNK, _D), jnp.float32),
        pltpu.VMEM((_CHUNK, _D), jnp.float32),
        pltpu.VMEM((_D, _CHUNK + 8), jnp.float32),
        pltpu.SemaphoreType.DMA,
        pltpu.SemaphoreType.DMA,
        pltpu.SemaphoreType.DMA,
    ],
    compiler_params=pltpu.CompilerParams(
        use_tc_tiling_on_sc=False, needs_layout_passes=False),
)
def _gather_rows(idx_hbm, table_hbm, out_hbm,
                 idx0, idx1, rows0, rows1, rows_t, gs0, gs1, ss):
    wid = lax.axis_index("s") * _NC + lax.axis_index("c")
    base = wid * _B_PER_W
    bufs = ((idx0, rows0, gs0), (idx1, rows1, gs1))
    iota = lax.iota(jnp.int32, _L)

    def load_idx(c, idx_v):
        pltpu.sync_copy(idx_hbm.at[pl.ds(base + c * _CHUNK, _CHUNK)], idx_v)

    def fire_gather(idx_v, rows_v, sem):
        pltpu.async_copy(table_hbm.at[idx_v], rows_v, sem)

    def wait_gather(idx_v, rows_v, sem):
        pltpu.make_async_copy(table_hbm.at[idx_v], rows_v, sem).wait()

    def out_slice(c):
        fl = base + c * _CHUNK
        return out_hbm.at[fl // _B_TOK, :, pl.ds(fl % _B_TOK, _CHUNK)]

    def transpose_chunk(rows_v):
        for d0 in range(0, _D, _L):
            dvec = iota + d0

            @plsc.parallel_loop(0, _CHUNK, unroll=8)
            def _r(r):
                vals = rows_v.at[r][pl.ds(d0, _L)]
                plsc.store_scatter(
                    rows_t, [dvec, jnp.full((_L,), r, jnp.int32)], vals)

    for b, (idx_v, rows_v, gsem) in enumerate(bufs):
        load_idx(b, idx_v)
        fire_gather(idx_v, rows_v, gsem)

    @pl.loop(0, _N_CHUNKS - 2, step=2)
    def _pair(i):
        for b, (idx_v, rows_v, gsem) in enumerate(bufs):
            c = i + b
            wait_gather(idx_v, rows_v, gsem)

            @pl.when(c > 0)
            def _():
                pltpu.make_async_copy(rows_t.at[:, pl.ds(0, _CHUNK)], out_slice(c), ss).wait()

            transpose_chunk(rows_v)
            pltpu.async_copy(rows_t.at[:, pl.ds(0, _CHUNK)], out_slice(c), ss)
            load_idx(c + 2, idx_v)
            fire_gather(idx_v, rows_v, gsem)

    for b, (idx_v, rows_v, gsem) in enumerate(bufs):
        c = _N_CHUNKS - 2 + b
        wait_gather(idx_v, rows_v, gsem)
        pltpu.make_async_copy(rows_t.at[:, pl.ds(0, _CHUNK)], out_slice(c), ss).wait()
        transpose_chunk(rows_v)
        pltpu.async_copy(rows_t.at[:, pl.ds(0, _CHUNK)], out_slice(c), ss)
    pltpu.make_async_copy(rows_t.at[:, pl.ds(0, _CHUNK)], out_slice(_N_CHUNKS - 1), ss).wait()


def kernel(token_ids, weight):
    idx = token_ids.T.ravel().astype(jnp.int32)
    out_p = _gather_rows(idx, weight)
    return jnp.transpose(out_p, (2, 0, 1))


# speedup vs baseline: 2.1251x; 1.0081x over previous
import functools

import jax
import jax.numpy as jnp
from jax import lax
from jax.experimental import pallas as pl
from jax.experimental.pallas import tpu as pltpu
from jax.experimental.pallas import tpu_sc as plsc

_B_TOK = 16384
_T = 50
_D = 64
_B = _B_TOK * _T
_NC = 2
_NS = 16
_NW = _NC * _NS
_B_PER_W = _B // _NW
_CHUNK = 256
_DW = 128
_N_CHUNKS = _B_PER_W // _CHUNK
_L = 16

_mesh = plsc.VectorSubcoreMesh(core_axis_name="c", subcore_axis_name="s")


@functools.partial(
    pl.kernel,
    mesh=_mesh,
    out_type=jax.ShapeDtypeStruct((_T, _D, _B_TOK), jnp.float32),
    scratch_types=[
        pltpu.VMEM((_CHUNK,), jnp.int32),
        pltpu.VMEM((_CHUNK,), jnp.int32),
        pltpu.VMEM((_CHUNK, _DW), jnp.float32),
        pltpu.VMEM((_CHUNK, _DW), jnp.float32),
        pltpu.VMEM((_D, _CHUNK + 8), jnp.float32),
        pltpu.SemaphoreType.DMA,
        pltpu.SemaphoreType.DMA,
        pltpu.SemaphoreType.DMA,
    ],
    compiler_params=pltpu.CompilerParams(
        use_tc_tiling_on_sc=False, needs_layout_passes=False),
)
def _gather_rows(idx_hbm, table_hbm, out_hbm,
                 idx0, idx1, rows0, rows1, rows_t, gs0, gs1, ss):
    wid = lax.axis_index("s") * _NC + lax.axis_index("c")
    base = wid * _B_PER_W
    bufs = ((idx0, rows0, gs0), (idx1, rows1, gs1))
    iota = lax.iota(jnp.int32, _L)

    def load_idx(c, idx_v):
        pltpu.sync_copy(idx_hbm.at[pl.ds(base + c * _CHUNK, _CHUNK)], idx_v)

    def fire_gather(idx_v, rows_v, sem):
        pltpu.async_copy(table_hbm.at[idx_v], rows_v, sem)

    def wait_gather(idx_v, rows_v, sem):
        pltpu.make_async_copy(table_hbm.at[idx_v], rows_v, sem).wait()

    def out_slice(c):
        fl = base + c * _CHUNK
        return out_hbm.at[fl // _B_TOK, :, pl.ds(fl % _B_TOK, _CHUNK)]

    def transpose_chunk(rows_v):
        for d0 in range(0, _D, _L):
            dvec = iota + d0

            @plsc.parallel_loop(0, _CHUNK, unroll=8)
            def _r(r):
                vals = rows_v.at[r][pl.ds(d0, _L)]
                plsc.store_scatter(
                    rows_t, [dvec, jnp.full((_L,), r, jnp.int32)], vals)

    for b, (idx_v, rows_v, gsem) in enumerate(bufs):
        load_idx(b, idx_v)
        fire_gather(idx_v, rows_v, gsem)

    @pl.loop(0, _N_CHUNKS - 2, step=2)
    def _pair(i):
        for b, (idx_v, rows_v, gsem) in enumerate(bufs):
            c = i + b
            wait_gather(idx_v, rows_v, gsem)

            @pl.when(c > 0)
            def _():
                pltpu.make_async_copy(rows_t.at[:, pl.ds(0, _CHUNK)], out_slice(c), ss).wait()

            transpose_chunk(rows_v)
            pltpu.async_copy(rows_t.at[:, pl.ds(0, _CHUNK)], out_slice(c), ss)
            load_idx(c + 2, idx_v)
            fire_gather(idx_v, rows_v, gsem)

    for b, (idx_v, rows_v, gsem) in enumerate(bufs):
        c = _N_CHUNKS - 2 + b
        wait_gather(idx_v, rows_v, gsem)
        pltpu.make_async_copy(rows_t.at[:, pl.ds(0, _CHUNK)], out_slice(c), ss).wait()
        transpose_chunk(rows_v)
        pltpu.async_copy(rows_t.at[:, pl.ds(0, _CHUNK)], out_slice(c), ss)
    pltpu.make_async_copy(rows_t.at[:, pl.ds(0, _CHUNK)], out_slice(_N_CHUNKS - 1), ss).wait()


def kernel(token_ids, weight):
    idx = token_ids.T.ravel().astype(jnp.int32)
    weight_p = jnp.pad(weight, ((0, 0), (0, _DW - _D)))
    out_p = _gather_rows(idx, weight_p)
    return jnp.transpose(out_p, (2, 0, 1))
